# Initial kernel scaffold; baseline (speedup 1.0000x reference)
import jax
import jax.numpy as jnp
from jax.experimental import pallas as pl


def kernel(x, Wg, bg, Wn, bn, W1, b1, W2, b2):
    raise NotImplementedError("write your pallas kernel here")


# speedup vs baseline: 1.2655x; 1.2655x over previous
import jax
import jax.numpy as jnp
from jax.experimental import pallas as pl
from jax.experimental.pallas import tpu as pltpu

_HI = jax.lax.Precision.HIGHEST


def _moe_body(x_ref, Wg_ref, bg_ref, Wn_ref, bn_ref, W1_ref, b1_ref,
              W2_ref, b2_ref, nz_ref, o_ref):
    e = pl.program_id(0)
    i = pl.program_id(1)
    BT = x_ref.shape[0]
    E = Wg_ref.shape[1]

    xt = x_ref[...]
    xb16 = xt.astype(jnp.bfloat16)
    hg = jnp.dot(xb16, Wg_ref[...].astype(jnp.bfloat16),
                 preferred_element_type=jnp.float32) + bg_ref[...]
    hn = jnp.dot(xb16, Wn_ref[...].astype(jnp.bfloat16),
                 preferred_element_type=jnp.float32) + bn_ref[...]
    sp = jnp.maximum(hn, 0.0) + jnp.log1p(jnp.exp(-jnp.abs(hn)))
    Hx = hg + nz_ref[...] * sp

    ii = jax.lax.broadcasted_iota(jnp.int32, (BT, E), 1)
    m1 = jnp.max(Hx, axis=1, keepdims=True)
    i1 = jnp.min(jnp.where(Hx == m1, ii, E), axis=1, keepdims=True)
    msk1 = ii == i1
    Hx2 = jnp.where(msk1, -jnp.inf, Hx)
    m2 = jnp.max(Hx2, axis=1, keepdims=True)
    i2 = jnp.min(jnp.where(Hx2 == m2, ii, E), axis=1, keepdims=True)
    msk2 = ii == i2
    s1 = 1.0 / (1.0 + jnp.exp(m2 - m1))
    s2 = 1.0 - s1
    w = s1 * msk1 + s2 * msk2
    wv = jnp.sum(jnp.where(ii == e, w, 0.0), axis=1, keepdims=True)

    xb = xt.astype(jnp.bfloat16)
    h = jnp.dot(xb, W1_ref[0].astype(jnp.bfloat16),
                preferred_element_type=jnp.float32) + b1_ref[0]
    h = jnp.maximum(h, 0.0)
    y = jnp.dot(h.astype(jnp.bfloat16), W2_ref[0].astype(jnp.bfloat16),
                preferred_element_type=jnp.float32) + b2_ref[0]
    contrib = wv * y

    sl = pl.ds(i * BT, BT)

    @pl.when(e == 0)
    def _():
        o_ref[sl, :] = contrib

    @pl.when(e > 0)
    def _():
        o_ref[sl, :] += contrib


def kernel(x, Wg, bg, Wn, bn, W1, b1, W2, b2):
    B, T, D = x.shape
    E = Wg.shape[1]
    FF = W1.shape[2]
    K = 2

    noise = jax.random.normal(jax.random.PRNGKey(42), shape=(B, T, E),
                              dtype=jnp.float32)
    x2 = x.reshape(T, D)
    nz2 = noise.reshape(T, E)
    b1r = b1.reshape(E, 1, FF)
    b2r = b2.reshape(E, 1, D)
    bgr = bg.reshape(1, E)
    bnr = bn.reshape(1, E)

    BT = 256
    NT = T // BT

    out = pl.pallas_call(
        _moe_body,
        grid=(E, NT),
        in_specs=[
            pl.BlockSpec((BT, D), lambda e, i: (i, 0)),
            pl.BlockSpec((D, E), lambda e, i: (0, 0)),
            pl.BlockSpec((1, E), lambda e, i: (0, 0)),
            pl.BlockSpec((D, E), lambda e, i: (0, 0)),
            pl.BlockSpec((1, E), lambda e, i: (0, 0)),
            pl.BlockSpec((1, D, FF), lambda e, i: (e, 0, 0)),
            pl.BlockSpec((1, 1, FF), lambda e, i: (e, 0, 0)),
            pl.BlockSpec((1, FF, D), lambda e, i: (e, 0, 0)),
            pl.BlockSpec((1, 1, D), lambda e, i: (e, 0, 0)),
            pl.BlockSpec((BT, E), lambda e, i: (i, 0)),
        ],
        out_specs=pl.BlockSpec((T, D), lambda e, i: (0, 0)),
        out_shape=jax.ShapeDtypeStruct((T, D), jnp.float32),
        compiler_params=pltpu.CompilerParams(
            dimension_semantics=("arbitrary", "arbitrary"),
        ),
    )(x2, Wg, bgr, Wn, bnr, W1, b1r, W2, b2r, nz2)

    return out.reshape(B, T, D)


# speedup vs baseline: 1.6819x; 1.3290x over previous
import jax
import jax.numpy as jnp
from jax import lax
from jax.experimental import pallas as pl
from jax.experimental.pallas import tpu as pltpu
from jax.experimental.pallas import tpu_sc as plsc

BT = 128
NT = 40
R = NT * BT
NW = 32
_BF = jnp.bfloat16


def _gate_body(x_ref, Wg_ref, bg_ref, Wn_ref, bn_ref, nz_ref,
               idx_ref, s_ref, pos_ref, cnt_ref):
    T, D = x_ref.shape
    E = Wg_ref.shape[1]
    xt = x_ref[...]
    xb = xt.astype(_BF)
    hg = jnp.dot(xb, Wg_ref[...].astype(_BF),
                 preferred_element_type=jnp.float32) + bg_ref[...]
    hn = jnp.dot(xb, Wn_ref[...].astype(_BF),
                 preferred_element_type=jnp.float32) + bn_ref[...]
    sp = jnp.maximum(hn, 0.0) + jnp.log1p(jnp.exp(-jnp.abs(hn)))
    Hx = hg + nz_ref[...] * sp

    ii = lax.broadcasted_iota(jnp.int32, (T, E), 1)
    m1 = jnp.max(Hx, axis=1, keepdims=True)
    i1 = jnp.min(jnp.where(Hx == m1, ii, E), axis=1, keepdims=True)
    msk1 = ii == i1
    Hx2 = jnp.where(msk1, -jnp.inf, Hx)
    m2 = jnp.max(Hx2, axis=1, keepdims=True)
    i2 = jnp.min(jnp.where(Hx2 == m2, ii, E), axis=1, keepdims=True)
    msk2 = ii == i2
    s1 = 1.0 / (1.0 + jnp.exp(m2 - m1))
    s2 = 1.0 - s1

    sel = (msk1 | msk2).astype(jnp.float32)
    SB = 256
    ri = lax.broadcasted_iota(jnp.int32, (SB, SB), 0)
    ci = lax.broadcasted_iota(jnp.int32, (SB, SB), 1)
    Lt = (ri > ci).astype(_BF)
    base = jnp.zeros((1, E), jnp.float32)
    blocks = []
    for b in range(T // SB):
        sb = sel[b * SB:(b + 1) * SB]
        blocks.append(base + jnp.dot(Lt, sb.astype(_BF),
                                     preferred_element_type=jnp.float32))
        base = base + jnp.sum(sb, axis=0, keepdims=True)
    cnt_before = jnp.concatenate(blocks, axis=0)

    pos0 = jnp.sum(jnp.where(msk1, cnt_before, 0.0), axis=1, keepdims=True)
    pos1 = jnp.sum(jnp.where(msk2, cnt_before, 0.0), axis=1, keepdims=True)

    idx_ref[...] = jnp.concatenate([i1, i2], axis=1)
    s_ref[...] = jnp.concatenate([s1, s2], axis=1)
    pos_ref[...] = jnp.concatenate([pos0, pos1], axis=1).astype(jnp.int32)
    cnt_ref[...] = jnp.broadcast_to(base, (8, E))


def _dispatch_body(x_h, pdst0_h, pdst1_h, xs_h, idx0_v, idx1_v, rows_v, sem):
    nc = plsc.get_sparse_core_info().num_cores
    wid = lax.axis_index("s") * nc + lax.axis_index("c")
    for c in range(2):
        off = wid * 64 + c * 32
        pltpu.sync_copy(pdst0_h.at[pl.ds(off, 32)], idx0_v)
        pltpu.sync_copy(pdst1_h.at[pl.ds(off, 32)], idx1_v)
        pltpu.sync_copy(x_h.at[pl.ds(off, 32)], rows_v)
        pltpu.async_copy(rows_v, xs_h.at[idx0_v], sem).wait()
        pltpu.async_copy(rows_v, xs_h.at[idx1_v], sem).wait()


def _ffn_body(g_ref, act_ref, xs_ref, W1_ref, b1_ref, W2_ref, b2_ref, ys_ref):
    r = pl.program_id(0)

    @pl.when(act_ref[r] == 1)
    def _():
        xb = xs_ref[...].astype(_BF)
        h = jnp.dot(xb, W1_ref[0].astype(_BF),
                    preferred_element_type=jnp.float32) + b1_ref[0]
        h = jnp.maximum(h, 0.0)
        ys_ref[...] = jnp.dot(h.astype(_BF), W2_ref[0].astype(_BF),
                              preferred_element_type=jnp.float32) + b2_ref[0]


def _combine_body(ys_h, pdst0_h, pdst1_h, s0m_h, s1m_h, res_h,
                  idx0_v, idx1_v, s0m_v, s1m_v, buf0_v, buf1_v, obuf_v, sem):
    nc = plsc.get_sparse_core_info().num_cores
    wid = lax.axis_index("s") * nc + lax.axis_index("c")
    D = 768
    for c in range(2):
        off = wid * 64 + c * 32
        pltpu.sync_copy(pdst0_h.at[pl.ds(off, 32)], idx0_v)
        pltpu.sync_copy(pdst1_h.at[pl.ds(off, 32)], idx1_v)
        pltpu.sync_copy(s0m_h.at[pl.ds(off, 32)], s0m_v)
        pltpu.sync_copy(s1m_h.at[pl.ds(off, 32)], s1m_v)
        pltpu.async_copy(ys_h.at[idx0_v], buf0_v, sem).wait()
        pltpu.async_copy(ys_h.at[idx1_v], buf1_v, sem).wait()

        def cj(j, cr):
            w0 = s0m_v[j, :]
            w1 = s1m_v[j, :]
            for v in range(D // 16):
                sl = pl.ds(v * 16, 16)
                obuf_v[j, sl] = w0 * buf0_v[j, sl] + w1 * buf1_v[j, sl]
            return cr
        lax.fori_loop(0, 32, cj, 0)
        pltpu.sync_copy(obuf_v, res_h.at[pl.ds(off, 32)])


def kernel(x, Wg, bg, Wn, bn, W1, b1, W2, b2):
    B, T, D = x.shape
    E = Wg.shape[1]
    FF = W1.shape[2]

    noise = jax.random.normal(jax.random.PRNGKey(42), shape=(B, T, E),
                              dtype=jnp.float32)
    x2 = x.reshape(T, D)
    nz2 = noise.reshape(T, E)
    b1r = b1.reshape(E, 1, FF)
    b2r = b2.reshape(E, 1, D)
    bgr = bg.reshape(1, E)
    bnr = bn.reshape(1, E)

    idx2, s2, pos2, cntf = pl.pallas_call(
        _gate_body,
        in_specs=[
            pl.BlockSpec((T, D), lambda: (0, 0)),
            pl.BlockSpec((D, E), lambda: (0, 0)),
            pl.BlockSpec((1, E), lambda: (0, 0)),
            pl.BlockSpec((D, E), lambda: (0, 0)),
            pl.BlockSpec((1, E), lambda: (0, 0)),
            pl.BlockSpec((T, E), lambda: (0, 0)),
        ],
        out_specs=[
            pl.BlockSpec((T, 2), lambda: (0, 0)),
            pl.BlockSpec((T, 2), lambda: (0, 0)),
            pl.BlockSpec((T, 2), lambda: (0, 0)),
            pl.BlockSpec((8, E), lambda: (0, 0)),
        ],
        out_shape=[
            jax.ShapeDtypeStruct((T, 2), jnp.int32),
            jax.ShapeDtypeStruct((T, 2), jnp.float32),
            jax.ShapeDtypeStruct((T, 2), jnp.int32),
            jax.ShapeDtypeStruct((8, E), jnp.float32),
        ],
    )(x2, Wg, bgr, Wn, bnr, nz2)

    cnt = cntf[0].astype(jnp.int32)
    pcnt = ((cnt + BT - 1) // BT) * BT
    pend = jnp.cumsum(pcnt)
    poff = pend - pcnt
    pdstm = poff[idx2] + pos2
    pdst0 = pdstm[:, 0] + 0
    pdst1 = pdstm[:, 1] + 0
    s0m = jnp.broadcast_to(s2[:, 0:1], (T, 16))
    s1m = jnp.broadcast_to(s2[:, 1:2], (T, 16))
    tile_start = jnp.arange(NT, dtype=jnp.int32) * BT
    gid = jnp.minimum(
        jnp.sum((tile_start[:, None] >= pend[None, :]).astype(jnp.int32),
                axis=1), E - 1).astype(jnp.int32)
    act = (tile_start < pend[E - 1]).astype(jnp.int32)

    mesh = plsc.VectorSubcoreMesh(core_axis_name="c", subcore_axis_name="s")

    xs = pl.kernel(
        _dispatch_body,
        out_type=jax.ShapeDtypeStruct((R, D), jnp.float32),
        mesh=mesh,
        scratch_types=[
            pltpu.VMEM((32,), jnp.int32),
            pltpu.VMEM((32,), jnp.int32),
            pltpu.VMEM((32, D), jnp.float32),
            pltpu.SemaphoreType.DMA,
        ],
    )(x2, pdst0, pdst1)

    ys = pl.pallas_call(
        _ffn_body,
        grid_spec=pltpu.PrefetchScalarGridSpec(
            num_scalar_prefetch=2,
            grid=(NT,),
            in_specs=[
                pl.BlockSpec((BT, D), lambda r, g, a: (r, 0)),
                pl.BlockSpec((1, D, FF), lambda r, g, a: (g[r], 0, 0)),
                pl.BlockSpec((1, 1, FF), lambda r, g, a: (g[r], 0, 0)),
                pl.BlockSpec((1, FF, D), lambda r, g, a: (g[r], 0, 0)),
                pl.BlockSpec((1, 1, D), lambda r, g, a: (g[r], 0, 0)),
            ],
            out_specs=pl.BlockSpec((BT, D), lambda r, g, a: (r, 0)),
        ),
        out_shape=jax.ShapeDtypeStruct((R, D), jnp.float32),
        compiler_params=pltpu.CompilerParams(
            dimension_semantics=("arbitrary",),
        ),
    )(gid, act, xs, W1, b1r, W2, b2r)

    res = pl.kernel(
        _combine_body,
        out_type=jax.ShapeDtypeStruct((T, D), jnp.float32),
        mesh=mesh,
        scratch_types=[
            pltpu.VMEM((32,), jnp.int32),
            pltpu.VMEM((32,), jnp.int32),
            pltpu.VMEM((32, 16), jnp.float32),
            pltpu.VMEM((32, 16), jnp.float32),
            pltpu.VMEM((32, D), jnp.float32),
            pltpu.VMEM((32, D), jnp.float32),
            pltpu.VMEM((32, D), jnp.float32),
            pltpu.SemaphoreType.DMA,
        ],
    )(ys, pdst0, pdst1, s0m, s1m)

    return res.reshape(B, T, D)


# speedup vs baseline: 1.8898x; 1.1237x over previous
import jax
import jax.numpy as jnp
from jax import lax
from jax.experimental import pallas as pl
from jax.experimental.pallas import tpu as pltpu
from jax.experimental.pallas import tpu_sc as plsc

BT = 256
NT = 25
R = NT * BT
NW = 32
_BF = jnp.bfloat16


def _gate_body(x_ref, Wgn_ref, bgn_ref, nz_ref,
               pdst_ref, s0m_ref, s1m_ref, pend_ref):
    T, D = x_ref.shape
    E = 8
    xt = x_ref[...]
    xb = xt.astype(_BF)
    hgn = jnp.dot(xb, Wgn_ref[...].astype(_BF),
                  preferred_element_type=jnp.float32) + bgn_ref[...]
    hg = hgn[:, :E]
    hn = hgn[:, E:]
    sp = jnp.maximum(hn, 0.0) + jnp.log1p(jnp.exp(-jnp.abs(hn)))
    Hx = hg + nz_ref[...] * sp

    ii = lax.broadcasted_iota(jnp.int32, (T, E), 1)
    m1 = jnp.max(Hx, axis=1, keepdims=True)
    i1 = jnp.min(jnp.where(Hx == m1, ii, E), axis=1, keepdims=True)
    msk1 = ii == i1
    Hx2 = jnp.where(msk1, -jnp.inf, Hx)
    m2 = jnp.max(Hx2, axis=1, keepdims=True)
    i2 = jnp.min(jnp.where(Hx2 == m2, ii, E), axis=1, keepdims=True)
    msk2 = ii == i2
    s1 = 1.0 / (1.0 + jnp.exp(m2 - m1))
    s2 = 1.0 - s1

    sel = (msk1 | msk2).astype(jnp.float32)
    SB = 256
    ri = lax.broadcasted_iota(jnp.int32, (SB, SB), 0)
    ci = lax.broadcasted_iota(jnp.int32, (SB, SB), 1)
    Lt = (ri > ci).astype(_BF)
    nb = T // SB
    bsums = [jnp.sum(sel[b * SB:(b + 1) * SB], axis=0, keepdims=True)
             for b in range(nb)]
    bases = [jnp.zeros((1, E), jnp.float32)]
    for b in range(nb - 1):
        bases.append(bases[b] + bsums[b])
    blocks = [bases[b] + jnp.dot(Lt, sel[b * SB:(b + 1) * SB].astype(_BF),
                                 preferred_element_type=jnp.float32)
              for b in range(nb)]
    cnt_before = jnp.concatenate(blocks, axis=0)
    cnt = bases[nb - 1] + bsums[nb - 1]

    pcnt = jnp.ceil(cnt * (1.0 / BT)) * BT
    pend = pcnt
    for sh in (1, 2, 4):
        pend = pend + jnp.concatenate(
            [jnp.zeros((1, sh), jnp.float32), pend[:, :E - sh]], axis=1)
    poff = pend - pcnt

    pos0 = jnp.sum(jnp.where(msk1, cnt_before + poff, 0.0),
                   axis=1, keepdims=True)
    pos1 = jnp.sum(jnp.where(msk2, cnt_before + poff, 0.0),
                   axis=1, keepdims=True)

    pdst_ref[...] = jnp.concatenate([pos0, pos1], axis=1).astype(jnp.int32)
    s0m_ref[...] = jnp.broadcast_to(s1, (T, 16))
    s1m_ref[...] = jnp.broadcast_to(s2, (T, 16))
    pend_ref[...] = jnp.broadcast_to(pend, (8, E))


def _dispatch_body(x_h, pdst0_h, pdst1_h, xs_h, idx0_v, idx1_v, rows_v, sem):
    nc = plsc.get_sparse_core_info().num_cores
    wid = lax.axis_index("s") * nc + lax.axis_index("c")
    for c in range(2):
        off = wid * 64 + c * 32
        pltpu.sync_copy(pdst0_h.at[pl.ds(off, 32)], idx0_v)
        pltpu.sync_copy(pdst1_h.at[pl.ds(off, 32)], idx1_v)
        pltpu.sync_copy(x_h.at[pl.ds(off, 32)], rows_v)
        h0 = pltpu.async_copy(rows_v, xs_h.at[idx0_v], sem)
        h1 = pltpu.async_copy(rows_v, xs_h.at[idx1_v], sem)
        h0.wait()
        h1.wait()


def _ffn_body(g_ref, act_ref, xs_ref, W1_ref, b1_ref, W2_ref, b2_ref, ys_ref):
    r = pl.program_id(0)

    @pl.when(act_ref[r] == 1)
    def _():
        xb = xs_ref[...].astype(_BF)
        h = jnp.dot(xb, W1_ref[0].astype(_BF),
                    preferred_element_type=jnp.float32) + b1_ref[0]
        h = jnp.maximum(h, 0.0)
        ys_ref[...] = jnp.dot(h.astype(_BF), W2_ref[0].astype(_BF),
                              preferred_element_type=jnp.float32) + b2_ref[0]


def _combine_body(ys_h, pdst0_h, pdst1_h, s0m_h, s1m_h, res_h,
                  idx0_v, idx1_v, s0m_v, s1m_v, buf0_v, buf1_v, obuf_v, sem):
    nc = plsc.get_sparse_core_info().num_cores
    wid = lax.axis_index("s") * nc + lax.axis_index("c")
    D = 768
    for c in range(2):
        off = wid * 64 + c * 32
        pltpu.sync_copy(pdst0_h.at[pl.ds(off, 32)], idx0_v)
        pltpu.sync_copy(pdst1_h.at[pl.ds(off, 32)], idx1_v)
        pltpu.sync_copy(s0m_h.at[pl.ds(off, 32)], s0m_v)
        pltpu.sync_copy(s1m_h.at[pl.ds(off, 32)], s1m_v)
        h0 = pltpu.async_copy(ys_h.at[idx0_v], buf0_v, sem)
        h1 = pltpu.async_copy(ys_h.at[idx1_v], buf1_v, sem)
        h0.wait()
        h1.wait()

        def cj(j, cr):
            w0 = s0m_v[j, :]
            w1 = s1m_v[j, :]
            for v in range(D // 16):
                sl = pl.ds(v * 16, 16)
                obuf_v[j, sl] = w0 * buf0_v[j, sl] + w1 * buf1_v[j, sl]
            return cr
        lax.fori_loop(0, 32, cj, 0)
        pltpu.sync_copy(obuf_v, res_h.at[pl.ds(off, 32)])


def kernel(x, Wg, bg, Wn, bn, W1, b1, W2, b2):
    B, T, D = x.shape
    E = Wg.shape[1]
    FF = W1.shape[2]

    noise = jax.random.normal(jax.random.PRNGKey(42), shape=(B, T, E),
                              dtype=jnp.float32)
    x2 = x.reshape(T, D)
    nz2 = noise.reshape(T, E)
    Wgn = jnp.concatenate([Wg, Wn], axis=1)
    bgn = jnp.concatenate([bg, bn]).reshape(1, 2 * E)
    b1r = b1.reshape(E, 1, FF)
    b2r = b2.reshape(E, 1, D)

    pdstm, s0m, s1m, pendb = pl.pallas_call(
        _gate_body,
        in_specs=[
            pl.BlockSpec((T, D), lambda: (0, 0)),
            pl.BlockSpec((D, 2 * E), lambda: (0, 0)),
            pl.BlockSpec((1, 2 * E), lambda: (0, 0)),
            pl.BlockSpec((T, E), lambda: (0, 0)),
        ],
        out_specs=[
            pl.BlockSpec((T, 2), lambda: (0, 0)),
            pl.BlockSpec((T, 16), lambda: (0, 0)),
            pl.BlockSpec((T, 16), lambda: (0, 0)),
            pl.BlockSpec((8, E), lambda: (0, 0)),
        ],
        out_shape=[
            jax.ShapeDtypeStruct((T, 2), jnp.int32),
            jax.ShapeDtypeStruct((T, 16), jnp.float32),
            jax.ShapeDtypeStruct((T, 16), jnp.float32),
            jax.ShapeDtypeStruct((8, E), jnp.float32),
        ],
    )(x2, Wgn, bgn, nz2)

    pend = pendb[0].astype(jnp.int32)
    pdst0 = pdstm[:, 0]
    pdst1 = pdstm[:, 1]
    tile_start = jnp.arange(NT, dtype=jnp.int32) * BT
    gid = jnp.minimum(
        jnp.sum((tile_start[:, None] >= pend[None, :]).astype(jnp.int32),
                axis=1), E - 1).astype(jnp.int32)
    act = (tile_start < pend[E - 1]).astype(jnp.int32)

    mesh = plsc.VectorSubcoreMesh(core_axis_name="c", subcore_axis_name="s")

    xs = pl.kernel(
        _dispatch_body,
        out_type=jax.ShapeDtypeStruct((R, D), jnp.float32),
        mesh=mesh,
        scratch_types=[
            pltpu.VMEM((32,), jnp.int32),
            pltpu.VMEM((32,), jnp.int32),
            pltpu.VMEM((32, D), jnp.float32),
            pltpu.SemaphoreType.DMA,
        ],
    )(x2, pdst0, pdst1)

    ys = pl.pallas_call(
        _ffn_body,
        grid_spec=pltpu.PrefetchScalarGridSpec(
            num_scalar_prefetch=2,
            grid=(NT,),
            in_specs=[
                pl.BlockSpec((BT, D), lambda r, g, a: (r, 0)),
                pl.BlockSpec((1, D, FF), lambda r, g, a: (g[r], 0, 0)),
                pl.BlockSpec((1, 1, FF), lambda r, g, a: (g[r], 0, 0)),
                pl.BlockSpec((1, FF, D), lambda r, g, a: (g[r], 0, 0)),
                pl.BlockSpec((1, 1, D), lambda r, g, a: (g[r], 0, 0)),
            ],
            out_specs=pl.BlockSpec((BT, D), lambda r, g, a: (r, 0)),
        ),
        out_shape=jax.ShapeDtypeStruct((R, D), jnp.float32),
        compiler_params=pltpu.CompilerParams(
            dimension_semantics=("arbitrary",),
        ),
    )(gid, act, xs, W1, b1r, W2, b2r)

    res = pl.kernel(
        _combine_body,
        out_type=jax.ShapeDtypeStruct((T, D), jnp.float32),
        mesh=mesh,
        scratch_types=[
            pltpu.VMEM((32,), jnp.int32),
            pltpu.VMEM((32,), jnp.int32),
            pltpu.VMEM((32, 16), jnp.float32),
            pltpu.VMEM((32, 16), jnp.float32),
            pltpu.VMEM((32, D), jnp.float32),
            pltpu.VMEM((32, D), jnp.float32),
            pltpu.VMEM((32, D), jnp.float32),
            pltpu.SemaphoreType.DMA,
        ],
    )(ys, pdst0, pdst1, s0m, s1m)

    return res.reshape(B, T, D)
